# baseline (speedup 1.0000x reference)
import jax
import jax.numpy as jnp
from jax import lax
from jax.experimental import pallas as pl
from jax.experimental.pallas import tpu as pltpu
from jax.experimental.pallas import tpu_sc as plsc

NC = 2
NS = 16
NW = NC * NS

NF = 16384
K = 26
D = 32
B = NF * K
SLABS = 4
NFS = NF // SLABS
BS = B // SLABS
BPW = BS // NW
C = 832
NCHUNK = BPW // C
FB = 64


def _make_body(slab):
    def _body(feat_hbm, table_hbm, out_hbm, idx_v, buf_v, g0, g1, w0, w1):
        wid = lax.axis_index("s") * NC + lax.axis_index("c")
        gbase = slab * BS + wid * BPW
        base = wid * BPW

        pltpu.sync_copy(feat_hbm.at[pl.ds(gbase, BPW)], idx_v)

        gsem = (g0, g1)
        wsem = (w0, w1)

        def _gather_args(c, b):
            return (table_hbm.at[idx_v.at[pl.ds(c * C, C)]], buf_v.at[b],
                    gsem[b])

        def _write_args(c, b):
            return (buf_v.at[b], out_hbm.at[pl.ds(base + c * C, C)], wsem[b])

        pltpu.async_copy(*_gather_args(0, 0))
        for c in range(NCHUNK):
            b = c & 1
            if c + 1 < NCHUNK:
                if c >= 1:
                    pltpu.make_async_copy(*_write_args(c - 1, 1 - b)).wait()
                pltpu.async_copy(*_gather_args(c + 1, 1 - b))
            pltpu.make_async_copy(*_gather_args(c, b)).wait()
            pltpu.async_copy(*_write_args(c, b))
        pltpu.make_async_copy(*_write_args(NCHUNK - 2, (NCHUNK - 2) & 1)).wait()
        pltpu.make_async_copy(*_write_args(NCHUNK - 1, (NCHUNK - 1) & 1)).wait()

    return _body


def _tc_body(x_ref, o_ref):
    o_ref[...] = x_ref[...].reshape(FB, K, D)


def _tc_relayout(slab_rows):
    return pl.pallas_call(
        _tc_body,
        grid=(NFS // FB,),
        in_specs=[pl.BlockSpec((FB * K, D), lambda i: (i, 0))],
        out_specs=pl.BlockSpec((FB, K, D), lambda i: (i, 0, 0)),
        out_shape=jax.ShapeDtypeStruct((NFS, K, D), jnp.float32),
    )(slab_rows)


@jax.jit
def _lookup(feats, table):
    mesh = plsc.VectorSubcoreMesh(core_axis_name="c", subcore_axis_name="s")
    outs = []
    for s in range(SLABS):
        o = pl.kernel(
            _make_body(s),
            out_type=jax.ShapeDtypeStruct((BS, D), jnp.float32),
            mesh=mesh,
            compiler_params=pltpu.CompilerParams(use_tc_tiling_on_sc=False),
            scratch_types=[
                pltpu.VMEM((BPW,), jnp.int32),
                pltpu.VMEM((2, C, D), jnp.float32),
                pltpu.SemaphoreType.DMA,
                pltpu.SemaphoreType.DMA,
                pltpu.SemaphoreType.DMA,
                pltpu.SemaphoreType.DMA,
            ],
        )(feats, table)
        outs.append(_tc_relayout(o))
    return jnp.concatenate(outs, axis=0)


def kernel(features, table):
    feats = features.reshape(-1).astype(jnp.int32)
    t = table.at[0].set(0.0)
    return _lookup(feats, t)


# speedup vs baseline: 2.1124x; 2.1124x over previous
import jax
import jax.numpy as jnp
from jax import lax
from jax.experimental import pallas as pl
from jax.experimental.pallas import tpu as pltpu
from jax.experimental.pallas import tpu_sc as plsc

NC = 2
NS = 16
NW = NC * NS

NF = 16384
K = 26
D = 32
B = NF * K
SLABS = 8
BS = B // SLABS
BPW = BS // NW
C = 416
NCHUNK = BPW // C


def _make_body(slab):
    def _body(feat_hbm, table_hbm, out_hbm, idx_v, buf_v, g0, g1, w0, w1):
        wid = lax.axis_index("s") * NC + lax.axis_index("c")
        gbase = slab * BS + wid * BPW
        base = wid * BPW

        pltpu.sync_copy(feat_hbm.at[pl.ds(gbase, BPW)], idx_v)

        gsem = (g0, g1)
        wsem = (w0, w1)

        def _gather_args(c, b):
            return (table_hbm.at[idx_v.at[pl.ds(c * C, C)]], buf_v.at[b],
                    gsem[b])

        def _write_args(c, b):
            return (buf_v.at[b], out_hbm.at[pl.ds(base + c * C, C)], wsem[b])

        pltpu.async_copy(*_gather_args(0, 0))
        for c in range(NCHUNK):
            b = c & 1
            if c + 1 < NCHUNK:
                if c >= 1:
                    pltpu.make_async_copy(*_write_args(c - 1, 1 - b)).wait()
                pltpu.async_copy(*_gather_args(c + 1, 1 - b))
            pltpu.make_async_copy(*_gather_args(c, b)).wait()
            pltpu.async_copy(*_write_args(c, b))
        pltpu.make_async_copy(*_write_args(NCHUNK - 2, (NCHUNK - 2) & 1)).wait()
        pltpu.make_async_copy(*_write_args(NCHUNK - 1, (NCHUNK - 1) & 1)).wait()

    return _body


@jax.jit
def _lookup(feats, table):
    mesh = plsc.VectorSubcoreMesh(core_axis_name="c", subcore_axis_name="s")
    outs = []
    for s in range(SLABS):
        o = pl.kernel(
            _make_body(s),
            out_type=jax.ShapeDtypeStruct((BS, D), jnp.float32),
            mesh=mesh,
            compiler_params=pltpu.CompilerParams(use_tc_tiling_on_sc=False),
            scratch_types=[
                pltpu.VMEM((BPW,), jnp.int32),
                pltpu.VMEM((2, C, D), jnp.float32),
                pltpu.SemaphoreType.DMA,
                pltpu.SemaphoreType.DMA,
                pltpu.SemaphoreType.DMA,
                pltpu.SemaphoreType.DMA,
            ],
        )(feats, table)
        outs.append(o.reshape(NF // SLABS, K, D))
    return jnp.concatenate(outs, axis=0)


def kernel(features, table):
    feats = features.reshape(-1).astype(jnp.int32)
    t = table.at[0].set(0.0)
    return _lookup(feats, t)


# speedup vs baseline: 2.1497x; 1.0176x over previous
import jax
import jax.numpy as jnp
from jax import lax
from jax.experimental import pallas as pl
from jax.experimental.pallas import tpu as pltpu
from jax.experimental.pallas import tpu_sc as plsc

NC = 2
NS = 16
NW = NC * NS

NF = 16384
K = 26
D = 32
B = NF * K
SLABS = 4
BS = B // SLABS
BPW = BS // NW
C = 832
NCHUNK = BPW // C


def _make_body(slab):
    def _body(feat_hbm, table_hbm, out_hbm, idx_v, buf_v, g0, g1, w0, w1):
        wid = lax.axis_index("s") * NC + lax.axis_index("c")
        gbase = slab * BS + wid * BPW
        base = wid * BPW

        pltpu.sync_copy(feat_hbm.at[pl.ds(gbase, BPW)], idx_v)

        gsem = (g0, g1)
        wsem = (w0, w1)

        def _gather_args(c, b):
            return (table_hbm.at[idx_v.at[pl.ds(c * C, C)]], buf_v.at[b],
                    gsem[b])

        def _write_args(c, b):
            return (buf_v.at[b], out_hbm.at[pl.ds(base + c * C, C)], wsem[b])

        pltpu.async_copy(*_gather_args(0, 0))
        for c in range(NCHUNK):
            b = c & 1
            if c + 1 < NCHUNK:
                if c >= 1:
                    pltpu.make_async_copy(*_write_args(c - 1, 1 - b)).wait()
                pltpu.async_copy(*_gather_args(c + 1, 1 - b))
            pltpu.make_async_copy(*_gather_args(c, b)).wait()
            pltpu.async_copy(*_write_args(c, b))
        pltpu.make_async_copy(*_write_args(NCHUNK - 2, (NCHUNK - 2) & 1)).wait()
        pltpu.make_async_copy(*_write_args(NCHUNK - 1, (NCHUNK - 1) & 1)).wait()

    return _body


@jax.jit
def _lookup(feats, table):
    mesh = plsc.VectorSubcoreMesh(core_axis_name="c", subcore_axis_name="s")
    outs = []
    for s in range(SLABS):
        o = pl.kernel(
            _make_body(s),
            out_type=jax.ShapeDtypeStruct((BS, D), jnp.float32),
            mesh=mesh,
            compiler_params=pltpu.CompilerParams(use_tc_tiling_on_sc=False),
            scratch_types=[
                pltpu.VMEM((BPW,), jnp.int32),
                pltpu.VMEM((2, C, D), jnp.float32),
                pltpu.SemaphoreType.DMA,
                pltpu.SemaphoreType.DMA,
                pltpu.SemaphoreType.DMA,
                pltpu.SemaphoreType.DMA,
            ],
        )(feats, table)
        outs.append(o.reshape(NF // SLABS, K, D))
    return jnp.concatenate(outs, axis=0)


def kernel(features, table):
    feats = features.reshape(-1).astype(jnp.int32)
    t = table.at[0].set(0.0)
    return _lookup(feats, t)
